# baseline (speedup 1.0000x reference)
import functools
import math

import jax
import jax.numpy as jnp
from jax import lax
from jax.experimental import pallas as pl
from jax.experimental.pallas import tpu as pltpu
from jax.experimental.pallas import tpu_sc as plsc

_NUM_CLASSES = 100000
_NUM_SAMPLED = 4096
_HIDDEN = 1024
_N = 2 * 2048
_K = _N + _NUM_SAMPLED
_S = _NUM_SAMPLED
_LOG_DENOM = math.log(_NUM_CLASSES + 1.0)

_NW = 32
_KW = _K // _NW
_CHUNK = 32
_WORDS = _CHUNK * _HIDDEN
_NCHUNK = _KW // _CHUNK


def _sc_gather(idx_flat, proj_flat, cls, bias):
    mesh = plsc.VectorSubcoreMesh(core_axis_name="c", subcore_axis_name="s")

    @functools.partial(
        pl.kernel,
        mesh=mesh,
        out_type=[
            jax.ShapeDtypeStruct((_K * _HIDDEN,), jnp.float32),
            jax.ShapeDtypeStruct((_K,), jnp.float32),
        ],
        scratch_types=[
            pltpu.VMEM((_WORDS,), jnp.int32),
            pltpu.VMEM((_WORDS,), jnp.float32),
            pltpu.VMEM((_KW,), jnp.int32),
            pltpu.VMEM((_KW,), jnp.float32),
            pltpu.SemaphoreType.DMA,
        ],
    )
    def gather_kernel(idx_hbm, proj_hbm, cls_hbm, bias_hbm,
                      w_hbm, biasg_hbm,
                      idx_v, row_v, cls_v, biasg_v, sem):
        wid = lax.axis_index("s") * 2 + lax.axis_index("c")
        base_k = wid * _KW

        pltpu.sync_copy(cls_hbm.at[pl.ds(base_k, _KW)], cls_v)
        pltpu.async_copy(bias_hbm.at[cls_v], biasg_v, sem).wait()
        pltpu.sync_copy(biasg_v, biasg_hbm.at[pl.ds(base_k, _KW)])

        def body(i, carry):
            off = (base_k + i * _CHUNK) * _HIDDEN
            pltpu.sync_copy(idx_hbm.at[pl.ds(off, _WORDS)], idx_v)
            pltpu.async_copy(proj_hbm.at[idx_v], row_v, sem).wait()
            pltpu.sync_copy(row_v, w_hbm.at[pl.ds(off, _WORDS)])
            return carry

        lax.fori_loop(0, _NCHUNK, body, 0)

    return gather_kernel(idx_flat, proj_flat, cls, bias)


def _log_corr(cf):
    return jnp.log(_NUM_SAMPLED * jnp.log((cf + 2.0) / (cf + 1.0)) / _LOG_DENOM)


def _loss_body(x_ref, tw_ref, sw_ref, bt_ref, bs_ref, lab_ref, samp_ref, out_ref):
    xb = x_ref[...]
    tw = tw_ref[...]
    sw = sw_ref[...]
    labels = lab_ref[...]
    sampled = samp_ref[...]
    bias_t = bt_ref[...]
    bias_s = bs_ref[...]

    logits_s = lax.dot_general(
        xb.astype(jnp.bfloat16), sw.astype(jnp.bfloat16),
        dimension_numbers=(((1,), (1,)), ((), ())),
        preferred_element_type=jnp.float32)

    true_logits = (jnp.sum(xb * tw, axis=1, keepdims=True)
                   + bias_t - _log_corr(labels.astype(jnp.float32)))
    logits_s = logits_s + bias_s - _log_corr(sampled.astype(jnp.float32))
    logits_s = jnp.where(labels == sampled, -1e9, logits_s)

    m = jnp.maximum(jnp.max(logits_s, axis=1, keepdims=True), true_logits)
    sumexp = (jnp.sum(jnp.exp(logits_s - m), axis=1, keepdims=True)
              + jnp.exp(true_logits - m))
    out_ref[...] = jnp.log(sumexp) + m - true_logits


_BN = 256


def _tc_loss(x2, w2, bt, bs, lab2, samp2):
    return pl.pallas_call(
        _loss_body,
        grid=(_N // _BN,),
        in_specs=[
            pl.BlockSpec((_BN, _HIDDEN), lambda i: (i, 0)),
            pl.BlockSpec((_BN, _HIDDEN), lambda i: (i, 0)),
            pl.BlockSpec((_S, _HIDDEN), lambda i: (1, 0)),
            pl.BlockSpec((_BN, 1), lambda i: (i, 0)),
            pl.BlockSpec((1, _S), lambda i: (0, 0)),
            pl.BlockSpec((_BN, 1), lambda i: (i, 0)),
            pl.BlockSpec((1, _S), lambda i: (0, 0)),
        ],
        out_specs=pl.BlockSpec((_BN, 1), lambda i: (i, 0)),
        out_shape=jax.ShapeDtypeStruct((_N, 1), jnp.float32),
    )(x2, w2, w2, bt, bs, lab2, samp2)


def kernel(y_true, input, projection, bias, sampled):
    labels = y_true.reshape(-1)
    x2 = input.reshape(_N, _HIDDEN)
    cls = jnp.concatenate([labels, sampled])
    idx = (cls[:, None]
           + (_NUM_CLASSES * jnp.arange(_HIDDEN, dtype=jnp.int32))[None, :])
    w_flat, bias_g = _sc_gather(idx.reshape(-1), projection.reshape(-1), cls, bias)
    w2 = w_flat.reshape(_K, _HIDDEN)
    loss = _tc_loss(x2, w2,
                    bias_g[:_N].reshape(_N, 1), bias_g[_N:].reshape(1, _S),
                    labels.reshape(_N, 1), sampled.reshape(1, _S))
    return loss.reshape(-1)


# speedup vs baseline: 10.1373x; 10.1373x over previous
import functools
import math

import jax
import jax.numpy as jnp
from jax import lax
from jax.experimental import pallas as pl
from jax.experimental.pallas import tpu as pltpu
from jax.experimental.pallas import tpu_sc as plsc

_NUM_CLASSES = 100000
_NUM_SAMPLED = 4096
_HIDDEN = 1024
_N = 2 * 2048
_K = _N + _NUM_SAMPLED
_S = _NUM_SAMPLED
_LOG_DENOM = math.log(_NUM_CLASSES + 1.0)

_NW = 32
_KW = _K // _NW
_GCHUNK = 64
_NCHUNK = _KW // _GCHUNK


def _sc_gather(weights, cls, bias):
    mesh = plsc.VectorSubcoreMesh(core_axis_name="c", subcore_axis_name="s")

    @functools.partial(
        pl.kernel,
        mesh=mesh,
        out_type=[
            jax.ShapeDtypeStruct((_K, _HIDDEN), jnp.float32),
            jax.ShapeDtypeStruct((_K,), jnp.float32),
        ],
        scratch_types=[
            pltpu.VMEM((_NCHUNK, _GCHUNK), jnp.int32),
            pltpu.VMEM((_GCHUNK, _HIDDEN), jnp.float32),
            pltpu.VMEM((_GCHUNK,), jnp.float32),
            pltpu.SemaphoreType.DMA,
        ],
    )
    def gather_kernel(w_hbm, cls_hbm, bias_hbm,
                      out_hbm, biasg_hbm,
                      cls_v, rows_v, biasg_v, sem):
        wid = lax.axis_index("s") * 2 + lax.axis_index("c")
        base_k = wid * _KW

        pltpu.sync_copy(cls_hbm.at[wid], cls_v)

        def body(i, carry):
            base = base_k + i * _GCHUNK
            pltpu.async_copy(w_hbm.at[cls_v.at[i]], rows_v, sem).wait()
            pltpu.sync_copy(rows_v, out_hbm.at[pl.ds(base, _GCHUNK)])
            pltpu.async_copy(bias_hbm.at[cls_v.at[i]], biasg_v, sem).wait()
            pltpu.sync_copy(biasg_v, biasg_hbm.at[pl.ds(base, _GCHUNK)])
            return carry

        lax.fori_loop(0, _NCHUNK, body, 0)

    return gather_kernel(weights, cls, bias)


def _log_corr(cf):
    return jnp.log(_NUM_SAMPLED * jnp.log((cf + 2.0) / (cf + 1.0)) / _LOG_DENOM)


def _loss_body(x_ref, tw_ref, sw_ref, bt_ref, bs_ref, lab_ref, samp_ref, out_ref):
    xb = x_ref[...]
    tw = tw_ref[...]
    sw = sw_ref[...]
    labels = lab_ref[...]
    sampled = samp_ref[...]
    bias_t = bt_ref[...]
    bias_s = bs_ref[...]

    logits_s = lax.dot_general(
        xb.astype(jnp.bfloat16), sw.astype(jnp.bfloat16),
        dimension_numbers=(((1,), (1,)), ((), ())),
        preferred_element_type=jnp.float32)

    true_logits = (jnp.sum(xb * tw, axis=1, keepdims=True)
                   + bias_t - _log_corr(labels.astype(jnp.float32)))
    logits_s = logits_s + bias_s - _log_corr(sampled.astype(jnp.float32))
    logits_s = jnp.where(labels == sampled, -1e9, logits_s)

    m = jnp.maximum(jnp.max(logits_s, axis=1, keepdims=True), true_logits)
    sumexp = (jnp.sum(jnp.exp(logits_s - m), axis=1, keepdims=True)
              + jnp.exp(true_logits - m))
    out_ref[...] = jnp.log(sumexp) + m - true_logits


_BN = 256


def _tc_loss(x2, w2, bt, bs, lab2, samp2):
    return pl.pallas_call(
        _loss_body,
        grid=(_N // _BN,),
        in_specs=[
            pl.BlockSpec((_BN, _HIDDEN), lambda i: (i, 0)),
            pl.BlockSpec((_BN, _HIDDEN), lambda i: (i, 0)),
            pl.BlockSpec((_S, _HIDDEN), lambda i: (1, 0)),
            pl.BlockSpec((_BN, 1), lambda i: (i, 0)),
            pl.BlockSpec((1, _S), lambda i: (0, 0)),
            pl.BlockSpec((_BN, 1), lambda i: (i, 0)),
            pl.BlockSpec((1, _S), lambda i: (0, 0)),
        ],
        out_specs=pl.BlockSpec((_BN, 1), lambda i: (i, 0)),
        out_shape=jax.ShapeDtypeStruct((_N, 1), jnp.float32),
    )(x2, w2, w2, bt, bs, lab2, samp2)


def kernel(y_true, input, projection, bias, sampled):
    labels = y_true.reshape(-1)
    x2 = input.reshape(_N, _HIDDEN)
    cls = jnp.concatenate([labels, sampled])
    weights = jnp.swapaxes(projection, 0, 1)
    w2, bias_g = _sc_gather(weights, cls.reshape(_NW, _NCHUNK, _GCHUNK), bias)
    loss = _tc_loss(x2, w2,
                    bias_g[:_N].reshape(_N, 1), bias_g[_N:].reshape(1, _S),
                    labels.reshape(_N, 1), sampled.reshape(1, _S))
    return loss.reshape(-1)
